# baseline (speedup 1.0000x reference)
import functools

import jax
import jax.numpy as jnp
from jax import lax
from jax.experimental import pallas as pl
from jax.experimental.pallas import tpu as pltpu
from jax.experimental.pallas import tpu_sc as plsc

_B = 32
_D = 128
_K = 8
_BN = 10000
_NSPLIT = 2
_BT = _BN * _NSPLIT


def _topk_body(nsteps, half, *refs):
    x_ref = refs[0]
    mem_refs = refs[1:1 + _NSPLIT]
    attn_ref, idx_ref, rv, ri = refs[1 + _NSPLIT:]
    i = pl.program_id(0)

    @pl.when(i == 0)
    def _init():
        rv[...] = jnp.full((_B, _K), -jnp.inf, dtype=jnp.float32)
        ri[...] = jnp.zeros((_B, _K), dtype=jnp.int32)

    xr = x_ref[...]
    x_n = xr * lax.rsqrt(jnp.sum(xr * xr, axis=1, keepdims=True))
    ones = jnp.ones((8, _D), dtype=jnp.float32)
    sims = []
    for mem_ref in mem_refs:
        m = mem_ref[...]
        nrm2 = lax.dot_general(ones, m * m, (((1,), (1,)), ((), ())),
                               preferred_element_type=jnp.float32)
        rn = lax.rsqrt(nrm2[0:1, :])
        sims.append(
            lax.dot_general(x_n, m, (((1,), (1,)), ((), ())),
                            preferred_element_type=jnp.float32) * rn)
    sim = jnp.concatenate(sims, axis=1)

    ii = lax.broadcasted_iota(jnp.int32, (_B, _BT), 1).astype(jnp.float32)
    neg_inf = jnp.float32(-jnp.inf)
    big = jnp.float32(3e7)
    s = sim
    bvs, bis = [], []
    for _ in range(_K):
        mx = jnp.max(s, axis=1, keepdims=True)
        eq = s == mx
        bvs.append(mx)
        bis.append(jnp.min(jnp.where(eq, ii, big), axis=1, keepdims=True))
        s = jnp.where(eq, neg_inf, s)
    bv = jnp.concatenate(bvs, axis=1)
    bif = jnp.concatenate(bis, axis=1)
    bk = jnp.zeros_like(bif)
    for t in range(1, _NSPLIT):
        bk = bk + jnp.where(bif >= t * _BN, 1.0, 0.0)
    bif = bif + bk * jnp.float32(half - _BN)
    bi = bif.astype(jnp.int32) + i * _BN

    cv = jnp.concatenate([rv[...], bv], axis=1)
    ci = jnp.concatenate([ri[...], bi], axis=1)
    big_i = jnp.int32(2**31 - 1)
    nvs, nis = [], []
    c = cv
    for _ in range(_K):
        mx = jnp.max(c, axis=1, keepdims=True)
        eq = c == mx
        nvs.append(mx)
        nis.append(jnp.min(jnp.where(eq, ci, big_i), axis=1, keepdims=True))
        c = jnp.where(eq, neg_inf, c)
    rv[...] = jnp.concatenate(nvs, axis=1)
    ri[...] = jnp.concatenate(nis, axis=1)

    @pl.when(i == nsteps - 1)
    def _done():
        v = rv[...] * 10.0
        e = jnp.exp(v - jnp.max(v, axis=1, keepdims=True))
        attn_ref[...] = e / jnp.sum(e, axis=1, keepdims=True)
        idx_ref[...] = ri[...]


def _topk_attn(x, memory):
    n = memory.shape[0]
    nsteps = n // _BT
    half = n // _NSPLIT
    mem_specs = [
        pl.BlockSpec((_BN, _D), lambda i, k=k: (i + k * nsteps, 0))
        for k in range(_NSPLIT)
    ]
    return pl.pallas_call(
        functools.partial(_topk_body, nsteps, half),
        grid=(nsteps,),
        in_specs=[
            pl.BlockSpec((_B, _D), lambda i: (0, 0)),
            *mem_specs,
        ],
        out_specs=[
            pl.BlockSpec((_B, _K), lambda i: (0, 0)),
            pl.BlockSpec((_B, _K), lambda i: (0, 0)),
        ],
        out_shape=[
            jax.ShapeDtypeStruct((_B, _K), jnp.float32),
            jax.ShapeDtypeStruct((_B, _K), jnp.int32),
        ],
        scratch_shapes=[
            pltpu.VMEM((_B, _K), jnp.float32),
            pltpu.VMEM((_B, _K), jnp.int32),
        ],
    )(x, *([memory] * _NSPLIT))


def _sc_gather_context(memory, idx_flat, w_flat):
    info = plsc.get_sparse_core_info()
    nc, ns = info.num_cores, info.num_subcores
    mesh = plsc.VectorSubcoreMesh(core_axis_name="c", subcore_axis_name="s")

    @functools.partial(
        pl.kernel,
        mesh=mesh,
        out_type=jax.ShapeDtypeStruct((_B, _D), jnp.float32),
        scratch_types=[
            pltpu.VMEM((_K,), jnp.int32),
            pltpu.VMEM((16,), jnp.float32),
            pltpu.VMEM((_K, _D), jnp.float32),
            pltpu.VMEM((_D,), jnp.float32),
            pltpu.SemaphoreType.DMA,
        ],
    )
    def gather_kernel(mem_hbm, idx_hbm, w_hbm, out_hbm,
                      idx_v, w_v, rows_v, out_v, sem):
        wid = lax.axis_index("s") * nc + lax.axis_index("c")
        base = wid * _K
        pltpu.sync_copy(idx_hbm.at[pl.ds(base, _K)], idx_v)
        pltpu.sync_copy(w_hbm.at[pl.ds(base, _K)], w_v.at[pl.ds(0, _K)])
        pltpu.async_copy(mem_hbm.at[idx_v], rows_v, sem).wait()
        wvec = w_v[...]
        for c in range(_D // 16):
            sl = pl.ds(c * 16, 16)
            acc = rows_v[0, sl] * wvec[0]
            for j in range(1, _K):
                acc = acc + rows_v[j, sl] * wvec[j]
            out_v[sl] = acc
        pltpu.sync_copy(out_v, out_hbm.at[wid])

    return gather_kernel(memory, idx_flat, w_flat)


def _final_body(x_ref, ctx_ref, w_ref, alpha_ref, o_ref):
    out = lax.dot_general(ctx_ref[...], w_ref[...], (((1,), (1,)), ((), ())),
                          preferred_element_type=jnp.float32)
    o_ref[...] = x_ref[...] + alpha_ref[0] * jnp.maximum(out, 0.0)


def _final(x, context, W, alpha):
    return pl.pallas_call(
        _final_body,
        in_specs=[
            pl.BlockSpec(memory_space=pltpu.VMEM),
            pl.BlockSpec(memory_space=pltpu.VMEM),
            pl.BlockSpec(memory_space=pltpu.VMEM),
            pl.BlockSpec(memory_space=pltpu.SMEM),
        ],
        out_shape=jax.ShapeDtypeStruct((_B, _D), jnp.float32),
    )(x, context, W, alpha)


@jax.jit
def kernel(x, memory, W, alpha):
    attn, idx = _topk_attn(x, memory)
    context = _sc_gather_context(memory, idx.reshape(-1), attn.reshape(-1))
    return _final(x, context, W, alpha)


# speedup vs baseline: 1.0045x; 1.0045x over previous
import functools

import jax
import jax.numpy as jnp
from jax import lax
from jax.experimental import pallas as pl
from jax.experimental.pallas import tpu as pltpu
from jax.experimental.pallas import tpu_sc as plsc

_B = 32
_D = 128
_K = 8
_BN = 10000
_NSPLIT = 2
_BT = _BN * _NSPLIT


def _topk_body(nsteps, half, *refs):
    x_ref = refs[0]
    mem_refs = refs[1:1 + _NSPLIT]
    attn_ref, idx_ref, rv, ri = refs[1 + _NSPLIT:]
    i = pl.program_id(0)

    @pl.when(i == 0)
    def _init():
        rv[...] = jnp.full((_B, _K), -jnp.inf, dtype=jnp.float32)
        ri[...] = jnp.zeros((_B, _K), dtype=jnp.int32)

    xr = x_ref[...]
    x_n = xr * lax.rsqrt(jnp.sum(xr * xr, axis=1, keepdims=True))
    ones = jnp.ones((8, _D), dtype=jnp.float32)
    sims = []
    for mem_ref in mem_refs:
        m = mem_ref[...]
        nrm2 = lax.dot_general(ones, m * m, (((1,), (1,)), ((), ())),
                               preferred_element_type=jnp.float32)
        rn = lax.rsqrt(nrm2[0:1, :])
        sims.append(
            lax.dot_general(x_n, m, (((1,), (1,)), ((), ())),
                            preferred_element_type=jnp.float32) * rn)
    sim = jnp.concatenate(sims, axis=1)

    ii = lax.broadcasted_iota(jnp.int32, (_B, _BT), 1).astype(jnp.float32)
    neg_inf = jnp.float32(-jnp.inf)
    big = jnp.float32(3e7)
    s = sim
    bvs, bis = [], []
    for _ in range(_K):
        mx = jnp.max(s, axis=1, keepdims=True)
        eq = s == mx
        bvs.append(mx)
        bis.append(jnp.min(jnp.where(eq, ii, big), axis=1, keepdims=True))
        s = jnp.where(eq, neg_inf, s)
    bv = jnp.concatenate(bvs, axis=1)
    bif = jnp.concatenate(bis, axis=1)
    bk = jnp.zeros_like(bif)
    for t in range(1, _NSPLIT):
        bk = bk + jnp.where(bif >= t * _BN, 1.0, 0.0)
    bif = bif + bk * jnp.float32(half - _BN)
    bi = bif.astype(jnp.int32) + i * _BN

    cv = jnp.concatenate([rv[...], bv], axis=1)
    ci = jnp.concatenate([ri[...], bi], axis=1)
    big_i = jnp.int32(2**31 - 1)
    nvs, nis = [], []
    c = cv
    for _ in range(_K):
        mx = jnp.max(c, axis=1, keepdims=True)
        eq = c == mx
        nvs.append(mx)
        nis.append(jnp.min(jnp.where(eq, ci, big_i), axis=1, keepdims=True))
        c = jnp.where(eq, neg_inf, c)
    rv[...] = jnp.concatenate(nvs, axis=1)
    ri[...] = jnp.concatenate(nis, axis=1)

    @pl.when(i == nsteps - 1)
    def _done():
        v = rv[...] * 10.0
        e = jnp.exp(v - jnp.max(v, axis=1, keepdims=True))
        attn_ref[...] = e / jnp.sum(e, axis=1, keepdims=True)
        idx_ref[...] = ri[...]


def _topk_attn(x, memory):
    n = memory.shape[0]
    nsteps = n // _BT
    half = n // _NSPLIT
    mem_specs = [
        pl.BlockSpec((_BN, _D), lambda i, k=k: (i + k * nsteps, 0))
        for k in range(_NSPLIT)
    ]
    return pl.pallas_call(
        functools.partial(_topk_body, nsteps, half),
        grid=(nsteps,),
        in_specs=[
            pl.BlockSpec((_B, _D), lambda i: (0, 0)),
            *mem_specs,
        ],
        out_specs=[
            pl.BlockSpec((_B, _K), lambda i: (0, 0)),
            pl.BlockSpec((_B, _K), lambda i: (0, 0)),
        ],
        out_shape=[
            jax.ShapeDtypeStruct((_B, _K), jnp.float32),
            jax.ShapeDtypeStruct((_B, _K), jnp.int32),
        ],
        scratch_shapes=[
            pltpu.VMEM((_B, _K), jnp.float32),
            pltpu.VMEM((_B, _K), jnp.int32),
        ],
    )(x, *([memory] * _NSPLIT))


def _sc_gather_context(memory, idx_flat, w_flat):
    info = plsc.get_sparse_core_info()
    nc, ns = info.num_cores, info.num_subcores
    mesh = plsc.VectorSubcoreMesh(core_axis_name="c", subcore_axis_name="s")

    @functools.partial(
        pl.kernel,
        mesh=mesh,
        out_type=jax.ShapeDtypeStruct((_B, _D), jnp.float32),
        scratch_types=[
            pltpu.VMEM((_K,), jnp.int32),
            pltpu.VMEM((16,), jnp.float32),
            pltpu.VMEM((_K, _D), jnp.float32),
            pltpu.VMEM((_D,), jnp.float32),
            pltpu.SemaphoreType.DMA,
            pltpu.SemaphoreType.DMA,
        ],
    )
    def gather_kernel(mem_hbm, idx_hbm, w_hbm, out_hbm,
                      idx_v, w_v, rows_v, out_v, sem, sem2):
        wid = lax.axis_index("s") * nc + lax.axis_index("c")
        base = wid * _K
        c1 = pltpu.async_copy(idx_hbm.at[pl.ds(base, _K)], idx_v, sem)
        c2 = pltpu.async_copy(w_hbm.at[pl.ds(base, _K)],
                              w_v.at[pl.ds(0, _K)], sem2)
        c1.wait()
        pltpu.async_copy(mem_hbm.at[idx_v], rows_v, sem).wait()
        c2.wait()
        wvec = w_v[...]
        for c in range(_D // 16):
            sl = pl.ds(c * 16, 16)
            acc = rows_v[0, sl] * wvec[0]
            for j in range(1, _K):
                acc = acc + rows_v[j, sl] * wvec[j]
            out_v[sl] = acc
        pltpu.sync_copy(out_v, out_hbm.at[wid])

    return gather_kernel(memory, idx_flat, w_flat)


def _final_body(x_ref, ctx_ref, w_ref, alpha_ref, o_ref):
    out = lax.dot_general(ctx_ref[...], w_ref[...], (((1,), (1,)), ((), ())),
                          preferred_element_type=jnp.float32)
    o_ref[...] = x_ref[...] + alpha_ref[0] * jnp.maximum(out, 0.0)


def _final(x, context, W, alpha):
    return pl.pallas_call(
        _final_body,
        in_specs=[
            pl.BlockSpec(memory_space=pltpu.VMEM),
            pl.BlockSpec(memory_space=pltpu.VMEM),
            pl.BlockSpec(memory_space=pltpu.VMEM),
            pl.BlockSpec(memory_space=pltpu.SMEM),
        ],
        out_shape=jax.ShapeDtypeStruct((_B, _D), jnp.float32),
    )(x, context, W, alpha)


@jax.jit
def kernel(x, memory, W, alpha):
    attn, idx = _topk_attn(x, memory)
    context = _sc_gather_context(memory, idx.reshape(-1), attn.reshape(-1))
    return _final(x, context, W, alpha)


# speedup vs baseline: 1.0349x; 1.0302x over previous
import functools

import jax
import jax.numpy as jnp
from jax import lax
from jax.experimental import pallas as pl
from jax.experimental.pallas import tpu as pltpu
from jax.experimental.pallas import tpu_sc as plsc

_B = 32
_D = 128
_K = 8
_BN = 20000
_NSPLIT = 1
_BT = _BN * _NSPLIT


def _topk_body(nsteps, half, *refs):
    x_ref = refs[0]
    mem_refs = refs[1:1 + _NSPLIT]
    attn_ref, idx_ref, rv, ri = refs[1 + _NSPLIT:]
    i = pl.program_id(0)

    @pl.when(i == 0)
    def _init():
        rv[...] = jnp.full((_B, _K), -jnp.inf, dtype=jnp.float32)
        ri[...] = jnp.zeros((_B, _K), dtype=jnp.int32)

    xr = x_ref[...]
    x_n = xr * lax.rsqrt(jnp.sum(xr * xr, axis=1, keepdims=True))
    ones = jnp.ones((8, _D), dtype=jnp.float32)
    sims = []
    for mem_ref in mem_refs:
        m = mem_ref[...]
        nrm2 = lax.dot_general(ones, m * m, (((1,), (1,)), ((), ())),
                               preferred_element_type=jnp.float32)
        rn = lax.rsqrt(nrm2[0:1, :])
        sims.append(
            lax.dot_general(x_n, m, (((1,), (1,)), ((), ())),
                            preferred_element_type=jnp.float32) * rn)
    sim = jnp.concatenate(sims, axis=1)

    ii = lax.broadcasted_iota(jnp.int32, (_B, _BT), 1).astype(jnp.float32)
    neg_inf = jnp.float32(-jnp.inf)
    big = jnp.float32(3e7)
    s = sim
    bvs, bis = [], []
    for _ in range(_K):
        mx = jnp.max(s, axis=1, keepdims=True)
        eq = s == mx
        bvs.append(mx)
        bis.append(jnp.min(jnp.where(eq, ii, big), axis=1, keepdims=True))
        s = jnp.where(eq, neg_inf, s)
    bv = jnp.concatenate(bvs, axis=1)
    bif = jnp.concatenate(bis, axis=1)
    bk = jnp.zeros_like(bif)
    for t in range(1, _NSPLIT):
        bk = bk + jnp.where(bif >= t * _BN, 1.0, 0.0)
    bif = bif + bk * jnp.float32(half - _BN)
    bi = bif.astype(jnp.int32) + i * _BN

    cv = jnp.concatenate([rv[...], bv], axis=1)
    ci = jnp.concatenate([ri[...], bi], axis=1)
    big_i = jnp.int32(2**31 - 1)
    nvs, nis = [], []
    c = cv
    for _ in range(_K):
        mx = jnp.max(c, axis=1, keepdims=True)
        eq = c == mx
        nvs.append(mx)
        nis.append(jnp.min(jnp.where(eq, ci, big_i), axis=1, keepdims=True))
        c = jnp.where(eq, neg_inf, c)
    rv[...] = jnp.concatenate(nvs, axis=1)
    ri[...] = jnp.concatenate(nis, axis=1)

    @pl.when(i == nsteps - 1)
    def _done():
        v = rv[...] * 10.0
        e = jnp.exp(v - jnp.max(v, axis=1, keepdims=True))
        attn_ref[...] = e / jnp.sum(e, axis=1, keepdims=True)
        idx_ref[...] = ri[...]


def _topk_attn(x, memory):
    n = memory.shape[0]
    nsteps = n // _BT
    half = n // _NSPLIT
    mem_specs = [
        pl.BlockSpec((_BN, _D), lambda i, k=k: (i + k * nsteps, 0))
        for k in range(_NSPLIT)
    ]
    return pl.pallas_call(
        functools.partial(_topk_body, nsteps, half),
        grid=(nsteps,),
        in_specs=[
            pl.BlockSpec((_B, _D), lambda i: (0, 0)),
            *mem_specs,
        ],
        out_specs=[
            pl.BlockSpec((_B, _K), lambda i: (0, 0)),
            pl.BlockSpec((_B, _K), lambda i: (0, 0)),
        ],
        out_shape=[
            jax.ShapeDtypeStruct((_B, _K), jnp.float32),
            jax.ShapeDtypeStruct((_B, _K), jnp.int32),
        ],
        scratch_shapes=[
            pltpu.VMEM((_B, _K), jnp.float32),
            pltpu.VMEM((_B, _K), jnp.int32),
        ],
    )(x, *([memory] * _NSPLIT))


def _sc_gather_context(memory, idx_flat, w_flat):
    info = plsc.get_sparse_core_info()
    nc, ns = info.num_cores, info.num_subcores
    mesh = plsc.VectorSubcoreMesh(core_axis_name="c", subcore_axis_name="s")

    @functools.partial(
        pl.kernel,
        mesh=mesh,
        out_type=jax.ShapeDtypeStruct((_B, _D), jnp.float32),
        scratch_types=[
            pltpu.VMEM((_K,), jnp.int32),
            pltpu.VMEM((16,), jnp.float32),
            pltpu.VMEM((_K, _D), jnp.float32),
            pltpu.VMEM((_D,), jnp.float32),
            pltpu.SemaphoreType.DMA,
            pltpu.SemaphoreType.DMA,
        ],
    )
    def gather_kernel(mem_hbm, idx_hbm, w_hbm, out_hbm,
                      idx_v, w_v, rows_v, out_v, sem, sem2):
        wid = lax.axis_index("s") * nc + lax.axis_index("c")
        base = wid * _K
        c1 = pltpu.async_copy(idx_hbm.at[pl.ds(base, _K)], idx_v, sem)
        c2 = pltpu.async_copy(w_hbm.at[pl.ds(base, _K)],
                              w_v.at[pl.ds(0, _K)], sem2)
        c1.wait()
        pltpu.async_copy(mem_hbm.at[idx_v], rows_v, sem).wait()
        c2.wait()
        wvec = w_v[...]
        for c in range(_D // 16):
            sl = pl.ds(c * 16, 16)
            acc = rows_v[0, sl] * wvec[0]
            for j in range(1, _K):
                acc = acc + rows_v[j, sl] * wvec[j]
            out_v[sl] = acc
        pltpu.sync_copy(out_v, out_hbm.at[wid])

    return gather_kernel(memory, idx_flat, w_flat)


def _final_body(x_ref, ctx_ref, w_ref, alpha_ref, o_ref):
    out = lax.dot_general(ctx_ref[...], w_ref[...], (((1,), (1,)), ((), ())),
                          preferred_element_type=jnp.float32)
    o_ref[...] = x_ref[...] + alpha_ref[0] * jnp.maximum(out, 0.0)


def _final(x, context, W, alpha):
    return pl.pallas_call(
        _final_body,
        in_specs=[
            pl.BlockSpec(memory_space=pltpu.VMEM),
            pl.BlockSpec(memory_space=pltpu.VMEM),
            pl.BlockSpec(memory_space=pltpu.VMEM),
            pl.BlockSpec(memory_space=pltpu.SMEM),
        ],
        out_shape=jax.ShapeDtypeStruct((_B, _D), jnp.float32),
    )(x, context, W, alpha)


@jax.jit
def kernel(x, memory, W, alpha):
    attn, idx = _topk_attn(x, memory)
    context = _sc_gather_context(memory, idx.reshape(-1), attn.reshape(-1))
    return _final(x, context, W, alpha)


# speedup vs baseline: 1.0389x; 1.0039x over previous
import functools

import jax
import jax.numpy as jnp
from jax import lax
from jax.experimental import pallas as pl
from jax.experimental.pallas import tpu as pltpu
from jax.experimental.pallas import tpu_sc as plsc

_B = 32
_D = 128
_K = 8
_BN = 25000
_NSPLIT = 1
_BT = _BN * _NSPLIT


def _topk_body(nsteps, half, *refs):
    x_ref = refs[0]
    mem_refs = refs[1:1 + _NSPLIT]
    attn_ref, idx_ref, rv, ri = refs[1 + _NSPLIT:]
    i = pl.program_id(0)

    @pl.when(i == 0)
    def _init():
        rv[...] = jnp.full((_B, _K), -jnp.inf, dtype=jnp.float32)
        ri[...] = jnp.zeros((_B, _K), dtype=jnp.int32)

    xr = x_ref[...]
    x_n = xr * lax.rsqrt(jnp.sum(xr * xr, axis=1, keepdims=True))
    ones = jnp.ones((8, _D), dtype=jnp.float32)
    sims = []
    for mem_ref in mem_refs:
        m = mem_ref[...]
        nrm2 = lax.dot_general(ones, m * m, (((1,), (1,)), ((), ())),
                               preferred_element_type=jnp.float32)
        rn = lax.rsqrt(nrm2[0:1, :])
        sims.append(
            lax.dot_general(x_n, m, (((1,), (1,)), ((), ())),
                            preferred_element_type=jnp.float32) * rn)
    sim = jnp.concatenate(sims, axis=1)

    ii = lax.broadcasted_iota(jnp.int32, (_B, _BT), 1).astype(jnp.float32)
    neg_inf = jnp.float32(-jnp.inf)
    big = jnp.float32(3e7)
    s = sim
    bvs, bis = [], []
    for _ in range(_K):
        mx = jnp.max(s, axis=1, keepdims=True)
        eq = s == mx
        bvs.append(mx)
        bis.append(jnp.min(jnp.where(eq, ii, big), axis=1, keepdims=True))
        s = jnp.where(eq, neg_inf, s)
    bv = jnp.concatenate(bvs, axis=1)
    bif = jnp.concatenate(bis, axis=1)
    bk = jnp.zeros_like(bif)
    for t in range(1, _NSPLIT):
        bk = bk + jnp.where(bif >= t * _BN, 1.0, 0.0)
    bif = bif + bk * jnp.float32(half - _BN)
    bi = bif.astype(jnp.int32) + i * _BN

    cv = jnp.concatenate([rv[...], bv], axis=1)
    ci = jnp.concatenate([ri[...], bi], axis=1)
    big_i = jnp.int32(2**31 - 1)
    nvs, nis = [], []
    c = cv
    for _ in range(_K):
        mx = jnp.max(c, axis=1, keepdims=True)
        eq = c == mx
        nvs.append(mx)
        nis.append(jnp.min(jnp.where(eq, ci, big_i), axis=1, keepdims=True))
        c = jnp.where(eq, neg_inf, c)
    rv[...] = jnp.concatenate(nvs, axis=1)
    ri[...] = jnp.concatenate(nis, axis=1)

    @pl.when(i == nsteps - 1)
    def _done():
        v = rv[...] * 10.0
        e = jnp.exp(v - jnp.max(v, axis=1, keepdims=True))
        attn_ref[...] = e / jnp.sum(e, axis=1, keepdims=True)
        idx_ref[...] = ri[...]


def _topk_attn(x, memory):
    n = memory.shape[0]
    nsteps = n // _BT
    half = n // _NSPLIT
    mem_specs = [
        pl.BlockSpec((_BN, _D), lambda i, k=k: (i + k * nsteps, 0))
        for k in range(_NSPLIT)
    ]
    return pl.pallas_call(
        functools.partial(_topk_body, nsteps, half),
        grid=(nsteps,),
        in_specs=[
            pl.BlockSpec((_B, _D), lambda i: (0, 0)),
            *mem_specs,
        ],
        out_specs=[
            pl.BlockSpec((_B, _K), lambda i: (0, 0)),
            pl.BlockSpec((_B, _K), lambda i: (0, 0)),
        ],
        out_shape=[
            jax.ShapeDtypeStruct((_B, _K), jnp.float32),
            jax.ShapeDtypeStruct((_B, _K), jnp.int32),
        ],
        scratch_shapes=[
            pltpu.VMEM((_B, _K), jnp.float32),
            pltpu.VMEM((_B, _K), jnp.int32),
        ],
    )(x, *([memory] * _NSPLIT))


def _sc_gather_context(memory, idx_flat, w_flat):
    info = plsc.get_sparse_core_info()
    nc, ns = info.num_cores, info.num_subcores
    mesh = plsc.VectorSubcoreMesh(core_axis_name="c", subcore_axis_name="s")

    @functools.partial(
        pl.kernel,
        mesh=mesh,
        out_type=jax.ShapeDtypeStruct((_B, _D), jnp.float32),
        scratch_types=[
            pltpu.VMEM((_K,), jnp.int32),
            pltpu.VMEM((16,), jnp.float32),
            pltpu.VMEM((_K, _D), jnp.float32),
            pltpu.VMEM((_D,), jnp.float32),
            pltpu.SemaphoreType.DMA,
            pltpu.SemaphoreType.DMA,
        ],
    )
    def gather_kernel(mem_hbm, idx_hbm, w_hbm, out_hbm,
                      idx_v, w_v, rows_v, out_v, sem, sem2):
        wid = lax.axis_index("s") * nc + lax.axis_index("c")
        base = wid * _K
        c1 = pltpu.async_copy(idx_hbm.at[pl.ds(base, _K)], idx_v, sem)
        c2 = pltpu.async_copy(w_hbm.at[pl.ds(base, _K)],
                              w_v.at[pl.ds(0, _K)], sem2)
        c1.wait()
        pltpu.async_copy(mem_hbm.at[idx_v], rows_v, sem).wait()
        c2.wait()
        wvec = w_v[...]
        for c in range(_D // 16):
            sl = pl.ds(c * 16, 16)
            acc = rows_v[0, sl] * wvec[0]
            for j in range(1, _K):
                acc = acc + rows_v[j, sl] * wvec[j]
            out_v[sl] = acc
        pltpu.sync_copy(out_v, out_hbm.at[wid])

    return gather_kernel(memory, idx_flat, w_flat)


def _final_body(x_ref, ctx_ref, w_ref, alpha_ref, o_ref):
    out = lax.dot_general(ctx_ref[...], w_ref[...], (((1,), (1,)), ((), ())),
                          preferred_element_type=jnp.float32)
    o_ref[...] = x_ref[...] + alpha_ref[0] * jnp.maximum(out, 0.0)


def _final(x, context, W, alpha):
    return pl.pallas_call(
        _final_body,
        in_specs=[
            pl.BlockSpec(memory_space=pltpu.VMEM),
            pl.BlockSpec(memory_space=pltpu.VMEM),
            pl.BlockSpec(memory_space=pltpu.VMEM),
            pl.BlockSpec(memory_space=pltpu.SMEM),
        ],
        out_shape=jax.ShapeDtypeStruct((_B, _D), jnp.float32),
    )(x, context, W, alpha)


@jax.jit
def kernel(x, memory, W, alpha):
    attn, idx = _topk_attn(x, memory)
    context = _sc_gather_context(memory, idx.reshape(-1), attn.reshape(-1))
    return _final(x, context, W, alpha)


# speedup vs baseline: 1.0571x; 1.0175x over previous
import functools

import jax
import jax.numpy as jnp
from jax import lax
from jax.experimental import pallas as pl
from jax.experimental.pallas import tpu as pltpu
from jax.experimental.pallas import tpu_sc as plsc

_B = 32
_D = 128
_K = 8
_BN = 25000
_NSPLIT = 1
_BT = _BN * _NSPLIT


def _topk_body(nsteps, half, *refs):
    x_ref = refs[0]
    mem_refs = refs[1:1 + _NSPLIT]
    attn_ref, idx_ref, rv, ri = refs[1 + _NSPLIT:]
    i = pl.program_id(0)

    @pl.when(i == 0)
    def _init():
        rv[...] = jnp.full((_B, _K), -jnp.inf, dtype=jnp.float32)
        ri[...] = jnp.zeros((_B, _K), dtype=jnp.int32)

    xr = x_ref[...]
    x_n = xr * lax.rsqrt(jnp.sum(xr * xr, axis=1, keepdims=True))
    ones = jnp.ones((8, _D), dtype=jnp.float32)
    sims = []
    for mem_ref in mem_refs:
        m = mem_ref[...]
        nrm2 = lax.dot_general(ones, m * m, (((1,), (1,)), ((), ())),
                               preferred_element_type=jnp.float32)
        rn = lax.rsqrt(nrm2[0:1, :])
        sims.append(
            lax.dot_general(x_n, m, (((1,), (1,)), ((), ())),
                            preferred_element_type=jnp.float32) * rn)
    sim = jnp.concatenate(sims, axis=1)

    ii = lax.broadcasted_iota(jnp.int32, (_B, _BT), 1).astype(jnp.float32)
    neg_inf = jnp.float32(-jnp.inf)
    big = jnp.float32(3e7)
    s = sim
    bvs, bis = [], []
    for _ in range(_K):
        mx = jnp.max(s, axis=1, keepdims=True)
        eq = s == mx
        bvs.append(mx)
        bis.append(jnp.min(jnp.where(eq, ii, big), axis=1, keepdims=True))
        s = jnp.where(eq, neg_inf, s)
    bv = jnp.concatenate(bvs, axis=1)
    bif = jnp.concatenate(bis, axis=1)
    bk = jnp.zeros_like(bif)
    for t in range(1, _NSPLIT):
        bk = bk + jnp.where(bif >= t * _BN, 1.0, 0.0)
    bif = bif + bk * jnp.float32(half - _BN)
    bi = bif.astype(jnp.int32) + i * _BN

    cv = jnp.concatenate([rv[...], bv], axis=1)
    ci = jnp.concatenate([ri[...], bi], axis=1)
    big_i = jnp.int32(2**31 - 1)
    nvs, nis = [], []
    c = cv
    for _ in range(_K):
        mx = jnp.max(c, axis=1, keepdims=True)
        eq = c == mx
        nvs.append(mx)
        nis.append(jnp.min(jnp.where(eq, ci, big_i), axis=1, keepdims=True))
        c = jnp.where(eq, neg_inf, c)
    rv[...] = jnp.concatenate(nvs, axis=1)
    ri[...] = jnp.concatenate(nis, axis=1)

    @pl.when(i == nsteps - 1)
    def _done():
        v = rv[...] * 10.0
        e = jnp.exp(v - jnp.max(v, axis=1, keepdims=True))
        attn_ref[...] = e / jnp.sum(e, axis=1, keepdims=True)
        idx_ref[...] = ri[...]


def _topk_attn(x, memory):
    n = memory.shape[0]
    nsteps = n // _BT
    half = n // _NSPLIT
    mem_specs = [
        pl.BlockSpec((_BN, _D), lambda i, k=k: (i + k * nsteps, 0))
        for k in range(_NSPLIT)
    ]
    return pl.pallas_call(
        functools.partial(_topk_body, nsteps, half),
        grid=(nsteps,),
        in_specs=[
            pl.BlockSpec((_B, _D), lambda i: (0, 0)),
            *mem_specs,
        ],
        out_specs=[
            pl.BlockSpec((_B, _K), lambda i: (0, 0)),
            pl.BlockSpec((_B, _K), lambda i: (0, 0)),
        ],
        out_shape=[
            jax.ShapeDtypeStruct((_B, _K), jnp.float32),
            jax.ShapeDtypeStruct((_B, _K), jnp.int32),
        ],
        scratch_shapes=[
            pltpu.VMEM((_B, _K), jnp.float32),
            pltpu.VMEM((_B, _K), jnp.int32),
        ],
    )(x, *([memory] * _NSPLIT))


def _sc_gather_context(memory, idx_flat, w_flat):
    info = plsc.get_sparse_core_info()
    nc, ns = info.num_cores, info.num_subcores
    mesh = plsc.VectorSubcoreMesh(core_axis_name="c", subcore_axis_name="s",
                                  num_cores=1)
    qpw = _B // ns

    @functools.partial(
        pl.kernel,
        mesh=mesh,
        out_type=jax.ShapeDtypeStruct((_B, _D), jnp.float32),
        scratch_types=[
            pltpu.VMEM((qpw * _K,), jnp.int32),
            pltpu.VMEM((qpw * _K + 8,), jnp.float32),
            pltpu.VMEM((qpw, _K, _D), jnp.float32),
            pltpu.VMEM((_D,), jnp.float32),
            pltpu.SemaphoreType.DMA,
            pltpu.SemaphoreType.DMA,
        ],
    )
    def gather_kernel(mem_hbm, idx_hbm, w_hbm, out_hbm,
                      idx_v, w_v, rows_v, out_v, sem, sem2):
        wid = lax.axis_index("s")
        base = wid * qpw * _K
        c1 = pltpu.async_copy(idx_hbm.at[pl.ds(base, qpw * _K)],
                              idx_v.at[...], sem)
        c2 = pltpu.async_copy(w_hbm.at[pl.ds(base, qpw * _K)],
                              w_v.at[pl.ds(0, qpw * _K)], sem2)
        c1.wait()
        gathers = [
            pltpu.async_copy(mem_hbm.at[idx_v.at[pl.ds(q * _K, _K)]],
                             rows_v.at[q], sem)
            for q in range(qpw)
        ]
        c2.wait()
        for q in range(qpw):
            gathers[q].wait()
            wvec = w_v[pl.ds(q * _K, 16)]
            for c in range(_D // 16):
                sl = pl.ds(c * 16, 16)
                acc = rows_v[q, 0, sl] * wvec[0]
                for j in range(1, _K):
                    acc = acc + rows_v[q, j, sl] * wvec[j]
                out_v[sl] = acc
            pltpu.sync_copy(out_v, out_hbm.at[wid * qpw + q])

    return gather_kernel(memory, idx_flat, w_flat)


def _final_body(x_ref, ctx_ref, w_ref, alpha_ref, o_ref):
    out = lax.dot_general(ctx_ref[...], w_ref[...], (((1,), (1,)), ((), ())),
                          preferred_element_type=jnp.float32)
    o_ref[...] = x_ref[...] + alpha_ref[0] * jnp.maximum(out, 0.0)


def _final(x, context, W, alpha):
    return pl.pallas_call(
        _final_body,
        in_specs=[
            pl.BlockSpec(memory_space=pltpu.VMEM),
            pl.BlockSpec(memory_space=pltpu.VMEM),
            pl.BlockSpec(memory_space=pltpu.VMEM),
            pl.BlockSpec(memory_space=pltpu.SMEM),
        ],
        out_shape=jax.ShapeDtypeStruct((_B, _D), jnp.float32),
    )(x, context, W, alpha)


@jax.jit
def kernel(x, memory, W, alpha):
    attn, idx = _topk_attn(x, memory)
    context = _sc_gather_context(memory, idx.reshape(-1), attn.reshape(-1))
    return _final(x, context, W, alpha)
